# baseline (speedup 1.0000x reference)
import functools
import jax
import jax.numpy as jnp
from jax import lax
from jax.experimental import pallas as pl
from jax.experimental.pallas import tpu as pltpu
from jax.experimental.pallas import tpu_sc as plsc

_NC = 2
_NS = 16
_NW = _NC * _NS
_TR = 16


def kernel(x, pos_table):
    B, M, D = x.shape
    xf = x.reshape(-1)
    pf = pos_table.reshape(-1)
    rows_w = M // _NW
    nt = rows_w // _TR
    te = _TR * D
    mesh = plsc.VectorSubcoreMesh(core_axis_name="c", subcore_axis_name="s")

    @functools.partial(
        pl.kernel,
        mesh=mesh,
        out_type=jax.ShapeDtypeStruct((B * M * D,), jnp.float32),
        scratch_types=[
            pltpu.VMEM((te,), jnp.float32),
            pltpu.VMEM((te,), jnp.float32),
            pltpu.VMEM((te,), jnp.float32),
        ],
    )
    def k(x_hbm, p_hbm, o_hbm, pbuf, xbuf, obuf):
        c = lax.axis_index("c")
        s = lax.axis_index("s")
        wid = s * _NC + c
        row0 = wid * rows_w

        def tile_body(t, _):
            pe0 = (row0 + t * _TR) * D
            pltpu.sync_copy(p_hbm.at[pl.ds(pe0, te)], pbuf)

            def batch_body(b, _):
                xe0 = b * (M * D) + pe0
                pltpu.sync_copy(x_hbm.at[pl.ds(xe0, te)], xbuf)

                def add_body(i, _):
                    sl = pl.ds(i * 16, 16)
                    obuf[sl] = xbuf[sl] + pbuf[sl]
                    return 0

                lax.fori_loop(0, te // 16, add_body, 0)
                pltpu.sync_copy(obuf, o_hbm.at[pl.ds(xe0, te)])
                return 0

            lax.fori_loop(0, B, batch_body, 0)
            return 0

        lax.fori_loop(0, nt, tile_body, 0)

    return k(xf, pf).reshape(B, M, D)


# speedup vs baseline: 1.3306x; 1.3306x over previous
import functools
import jax
import jax.numpy as jnp
from jax import lax
from jax.experimental import pallas as pl
from jax.experimental.pallas import tpu as pltpu
from jax.experimental.pallas import tpu_sc as plsc

_NC = 2
_NS = 16
_NW = _NC * _NS
_TR = 16
_UNROLL = 8


def kernel(x, pos_table):
    B, M, D = x.shape
    xf = x.reshape(-1)
    pf = pos_table.reshape(-1)
    rows_w = M // _NW
    nt = rows_w // _TR
    te = _TR * D
    md = M * D
    mesh = plsc.VectorSubcoreMesh(core_axis_name="c", subcore_axis_name="s")

    @functools.partial(
        pl.kernel,
        mesh=mesh,
        out_type=jax.ShapeDtypeStruct((B * M * D,), jnp.float32),
        scratch_types=[
            pltpu.VMEM((te,), jnp.float32),
            pltpu.VMEM((te,), jnp.float32),
        ],
    )
    def k(x_hbm, p_hbm, o_hbm, pbuf, obuf):
        c = lax.axis_index("c")
        s = lax.axis_index("s")
        wid = s * _NC + c
        row0 = wid * rows_w

        def tile_body(t, _):
            pe0 = (row0 + t * _TR) * D
            pltpu.sync_copy(p_hbm.at[pl.ds(pe0, te)], pbuf)

            def batch_body(b, _):
                xe0 = b * md + pe0
                pltpu.sync_copy(x_hbm.at[pl.ds(xe0, te)], obuf)

                def add_body(i, _):
                    base = i * (16 * _UNROLL)
                    for u in range(_UNROLL):
                        sl = pl.ds(base + u * 16, 16)
                        plsc.addupdate(obuf.at[sl], pbuf[sl])
                    return 0

                lax.fori_loop(0, te // (16 * _UNROLL), add_body, 0)
                pltpu.sync_copy(obuf, o_hbm.at[pl.ds(xe0, te)])
                return 0

            lax.fori_loop(0, B, batch_body, 0)
            return 0

        lax.fori_loop(0, nt, tile_body, 0)

    return k(xf, pf).reshape(B, M, D)


# speedup vs baseline: 1.6005x; 1.2029x over previous
import functools
import jax
import jax.numpy as jnp
from jax import lax
from jax.experimental import pallas as pl
from jax.experimental.pallas import tpu as pltpu
from jax.experimental.pallas import tpu_sc as plsc

_NC = 2
_NS = 16
_NW = _NC * _NS
_TR = 32
_UNROLL = 8


def kernel(x, pos_table):
    B, M, D = x.shape
    xf = x.reshape(-1)
    pf = pos_table.reshape(-1)
    rows_w = M // _NW
    nt = rows_w // _TR
    te = _TR * D
    md = M * D
    mesh = plsc.VectorSubcoreMesh(core_axis_name="c", subcore_axis_name="s")

    @functools.partial(
        pl.kernel,
        mesh=mesh,
        out_type=jax.ShapeDtypeStruct((B * M * D,), jnp.float32),
        scratch_types=[
            pltpu.VMEM((te,), jnp.float32),
            pltpu.VMEM((te,), jnp.float32),
            pltpu.VMEM((te,), jnp.float32),
            pltpu.SemaphoreType.DMA,
            pltpu.SemaphoreType.DMA,
            pltpu.SemaphoreType.DMA,
            pltpu.SemaphoreType.DMA,
        ],
    )
    def k(x_hbm, p_hbm, o_hbm, pbuf, ob0, ob1, si0, si1, so0, so1):
        c = lax.axis_index("c")
        s = lax.axis_index("s")
        wid = s * _NC + c
        row0 = wid * rows_w
        ob = (ob0, ob1)
        si = (si0, si1)
        so = (so0, so1)

        def add_tile(buf):
            def add_body(i, _):
                base = i * (16 * _UNROLL)
                for u in range(_UNROLL):
                    sl = pl.ds(base + u * 16, 16)
                    plsc.addupdate(buf.at[sl], pbuf[sl])
                return 0

            lax.fori_loop(0, te // (16 * _UNROLL), add_body, 0)

        def tile_body(t, _):
            pe0 = (row0 + t * _TR) * D
            pltpu.sync_copy(p_hbm.at[pl.ds(pe0, te)], pbuf)
            din = [None] * B
            dout = [None] * B
            din[0] = pltpu.async_copy(x_hbm.at[pl.ds(pe0, te)], ob[0], si[0])
            for b in range(B):
                if b + 1 < B:
                    if b >= 1:
                        dout[b - 1].wait()
                    nxt = (b + 1) * md + pe0
                    din[b + 1] = pltpu.async_copy(
                        x_hbm.at[pl.ds(nxt, te)], ob[(b + 1) % 2], si[(b + 1) % 2]
                    )
                din[b].wait()
                add_tile(ob[b % 2])
                dout[b] = pltpu.async_copy(
                    ob[b % 2], o_hbm.at[pl.ds(b * md + pe0, te)], so[b % 2]
                )
            dout[B - 2].wait()
            dout[B - 1].wait()
            return 0

        lax.fori_loop(0, nt, tile_body, 0)

    return k(xf, pf).reshape(B, M, D)


# speedup vs baseline: 4.1111x; 2.5686x over previous
import functools
import jax
import jax.numpy as jnp
from jax import lax
from jax.experimental import pallas as pl
from jax.experimental.pallas import tpu as pltpu
from jax.experimental.pallas import tpu_sc as plsc

_NC = 2
_NS = 16
_NW = _NC * _NS
_TR = 32


def kernel(x, pos_table):
    B, M, D = x.shape
    x2 = x.reshape(B * M, D)
    rows_w = M // _NW
    nt = rows_w // _TR
    mesh = plsc.VectorSubcoreMesh(core_axis_name="c", subcore_axis_name="s")

    @functools.partial(
        pl.kernel,
        mesh=mesh,
        out_type=jax.ShapeDtypeStruct((B * M, D), jnp.float32),
        scratch_types=[
            pltpu.VMEM((_TR, D), jnp.float32),
            pltpu.VMEM((_TR, D), jnp.float32),
            pltpu.VMEM((_TR, D), jnp.float32),
            pltpu.SemaphoreType.DMA,
            pltpu.SemaphoreType.DMA,
            pltpu.SemaphoreType.DMA,
            pltpu.SemaphoreType.DMA,
        ],
    )
    def k(x_hbm, p_hbm, o_hbm, pbuf, ob0, ob1, si0, si1, so0, so1):
        c = lax.axis_index("c")
        s = lax.axis_index("s")
        wid = s * _NC + c
        row0 = wid * rows_w
        ob = (ob0, ob1)
        si = (si0, si1)
        so = (so0, so1)

        def add_tile(buf):
            def row_body(r, _):
                for j in range(D // 16):
                    sl = pl.ds(j * 16, 16)
                    plsc.addupdate(buf.at[r, sl], pbuf[r, sl])
                return 0

            lax.fori_loop(0, _TR, row_body, 0)

        def tile_body(t, _):
            prow = row0 + t * _TR
            pltpu.sync_copy(p_hbm.at[pl.ds(prow, _TR)], pbuf)
            din = [None] * B
            dout = [None] * B
            din[0] = pltpu.async_copy(x_hbm.at[pl.ds(prow, _TR)], ob[0], si[0])
            for b in range(B):
                if b + 1 < B:
                    if b >= 1:
                        dout[b - 1].wait()
                    nxt = (b + 1) * M + prow
                    din[b + 1] = pltpu.async_copy(
                        x_hbm.at[pl.ds(nxt, _TR)], ob[(b + 1) % 2], si[(b + 1) % 2]
                    )
                din[b].wait()
                add_tile(ob[b % 2])
                dout[b] = pltpu.async_copy(
                    ob[b % 2], o_hbm.at[pl.ds(b * M + prow, _TR)], so[b % 2]
                )
            dout[B - 2].wait()
            dout[B - 1].wait()
            return 0

        lax.fori_loop(0, nt, tile_body, 0)

    return k(x2, pos_table).reshape(B, M, D)
